# Initial kernel scaffold; baseline (speedup 1.0000x reference)
import jax
import jax.numpy as jnp
from jax.experimental import pallas as pl


def kernel(coords, feats, W1, g1, b1, W2, g2, b2, W3, g3, b3, W4, g4, b4, W5, g5, b5, Wf5, bf5, gf5, betaf5, Wf6, bf6, gf6, betaf6, Wf7, bf7, gf7, betaf7, Wf8, bf8, gf8, betaf8, Wsem, bsem):
    raise NotImplementedError("write your pallas kernel here")


# speedup vs baseline: 8.3052x; 8.3052x over previous
import functools

import jax
import jax.numpy as jnp
from jax import lax
from jax.experimental import pallas as pl
from jax.experimental.pallas import tpu as pltpu
from jax.experimental.pallas import tpu_sc as plsc

_pc = pl.pallas_call

N = 4096
KNN = 20
BLK = 256
NBLK = N // BLK
EPS = 1e-5


def _lrelu(h):
    return jnp.where(h >= 0, h, 0.2 * h)


def _topk_body(x_blk, x_all, xt_ref, out_ref):
    xb = x_blk[...]
    xt = xt_ref[...]
    c = xt.shape[0]
    if c == 4:
        xbr = xb.astype(jnp.bfloat16).astype(jnp.float32)
        xtr = xt.astype(jnp.bfloat16).astype(jnp.float32)
        ip = xbr[:, 0:1] * xtr[0:1, :]
        for cc in range(1, 4):
            ip = ip + xbr[:, cc : cc + 1] * xtr[cc : cc + 1, :]
    else:
        xa = x_all[...]
        ip = lax.dot_general(xb, xa, (((1,), (1,)), ((), ())),
                             preferred_element_type=jnp.float32)
    xx_row = jnp.sum(xt * xt, axis=0, keepdims=True)
    xx_blk = jnp.sum(xb * xb, axis=1, keepdims=True)
    pd = (2.0 * ip - xx_blk) - xx_row
    iota = lax.broadcasted_iota(jnp.int32, (BLK, N), 1)
    for j in range(KNN):
        m = jnp.max(pd, axis=1, keepdims=True)
        cand = jnp.where(pd == m, iota, jnp.int32(N))
        sel = jnp.min(cand, axis=1, keepdims=True)
        out_ref[:, j : j + 1] = sel
        pd = jnp.where(iota == sel, -jnp.inf, pd)


@functools.lru_cache(maxsize=None)
def _topk_call(c):
    return _pc(
        _topk_body,
        grid=(NBLK,),
        in_specs=[
            pl.BlockSpec((BLK, c), lambda i: (i, 0)),
            pl.BlockSpec((N, c), lambda i: (0, 0)),
            pl.BlockSpec((c, N), lambda i: (0, 0)),
        ],
        out_specs=pl.BlockSpec((BLK, 32), lambda i: (i, 0)),
        out_shape=jax.ShapeDtypeStruct((N, 32), jnp.int32),
    )


CP = 128


@functools.lru_cache(maxsize=None)
def _sc_gather_call():
    nc, ns = 2, 16
    nw = nc * ns
    nb = N // nw
    npairs = nb // 2

    mesh = plsc.VectorSubcoreMesh(core_axis_name="c", subcore_axis_name="s")

    @functools.partial(
        pl.kernel,
        mesh=mesh,
        out_type=jax.ShapeDtypeStruct((N * KNN, CP), jnp.float32),
        scratch_types=[
            pltpu.VMEM((nb * KNN,), jnp.int32),
            pltpu.VMEM((2 * KNN, CP), jnp.float32),
            pltpu.SemaphoreType.DMA,
        ],
    )
    def k(idx_hbm, x_hbm, out_hbm, idx_v, buf, sem):
        wid = lax.axis_index("s") * nc + lax.axis_index("c")
        base = wid * (nb * KNN)
        pltpu.sync_copy(idx_hbm.at[pl.ds(base, nb * KNN)], idx_v)

        def pair(p, carry):
            off = pl.multiple_of(p * (2 * KNN), 8)
            pltpu.async_copy(
                x_hbm.at[idx_v.at[pl.ds(off, 2 * KNN)]], buf, sem
            ).wait()
            pltpu.sync_copy(
                buf, out_hbm.at[pl.ds(base + p * (2 * KNN), 2 * KNN)]
            )
            return carry

        lax.fori_loop(0, npairs, pair, 0)

    return k


BH = 128
NBH = N // BH


def _edge_body(g_ref, x_ref, wt_ref, hmax_ref, hmin_ref, st_ref):
    i = pl.program_id(0)
    xb = x_ref[...]
    c = xb.shape[1]
    o = wt_ref.shape[1]
    gd = g_ref[...][:, :c]
    d3 = gd.reshape(BH, KNN, c) - xb[:, None, :]
    d2 = d3.reshape(BH * KNN, c)
    x3 = jnp.broadcast_to(xb[:, None, :], (BH, KNN, c))
    x2 = x3.reshape(BH * KNN, c)
    if c == 4:
        d2r = d2.astype(jnp.bfloat16).astype(jnp.float32)
        x2r = x2.astype(jnp.bfloat16).astype(jnp.float32)
        wtr = wt_ref[...].astype(jnp.bfloat16).astype(jnp.float32)
        h2 = d2r[:, 0:1] * wtr[0:1, :]
        for cc in range(1, 4):
            h2 = h2 + d2r[:, cc : cc + 1] * wtr[cc : cc + 1, :]
        for cc in range(4):
            h2 = h2 + x2r[:, cc : cc + 1] * wtr[4 + cc : 5 + cc, :]
    else:
        f2 = jnp.concatenate([d2, x2], axis=1)
        h2 = lax.dot_general(f2, wt_ref[...], (((1,), (0,)), ((), ())),
                             preferred_element_type=jnp.float32)
    h3 = h2.reshape(BH, KNN, o)
    hmax_ref[...] = jnp.max(h3, axis=1)
    hmin_ref[...] = jnp.min(h3, axis=1)
    part = jnp.concatenate(
        [
            jnp.sum(h2, axis=0, keepdims=True),
            jnp.sum(h2 * h2, axis=0, keepdims=True),
            jnp.zeros((6, o), jnp.float32),
        ],
        0,
    )

    @pl.when(i == 0)
    def _():
        st_ref[...] = jnp.zeros_like(st_ref)

    st_ref[...] += part


@functools.lru_cache(maxsize=None)
def _edge_call(c, o):
    return _pc(
        _edge_body,
        grid=(NBH,),
        in_specs=[
            pl.BlockSpec((BH * KNN, CP), lambda i: (i, 0)),
            pl.BlockSpec((BH, c), lambda i: (i, 0)),
            pl.BlockSpec((2 * c, o), lambda i: (0, 0)),
        ],
        out_specs=(
            pl.BlockSpec((BH, o), lambda i: (i, 0)),
            pl.BlockSpec((BH, o), lambda i: (i, 0)),
            pl.BlockSpec((8, o), lambda i: (0, 0)),
        ),
        out_shape=(
            jax.ShapeDtypeStruct((N, o), jnp.float32),
            jax.ShapeDtypeStruct((N, o), jnp.float32),
            jax.ShapeDtypeStruct((8, o), jnp.float32),
        ),
    )


def _fin_body(stats_ref, gb_ref, hmax_ref, hmin_ref, out_ref):
    st = stats_ref[...]
    denom = 1.0 / (N * KNN)
    mean = st[0:1] * denom
    var = st[1:2] * denom - mean * mean
    a = gb_ref[0:1] * lax.rsqrt(var + EPS)
    cc = gb_ref[1:2] - a * mean
    pre = jnp.where(a >= 0, hmax_ref[...], hmin_ref[...])
    out_ref[...] = _lrelu(a * pre + cc)


@functools.lru_cache(maxsize=None)
def _fin_call(o):
    return _pc(
        _fin_body,
        grid=(NBLK,),
        in_specs=[
            pl.BlockSpec((8, o), lambda i: (0, 0)),
            pl.BlockSpec((8, o), lambda i: (0, 0)),
            pl.BlockSpec((BLK, o), lambda i: (i, 0)),
            pl.BlockSpec((BLK, o), lambda i: (i, 0)),
        ],
        out_specs=pl.BlockSpec((BLK, o), lambda i: (i, 0)),
        out_shape=jax.ShapeDtypeStruct((N, o), jnp.float32),
    )


def _h1_body(x1_ref, x2_ref, x3_ref, x4_ref, w5_ref, h5_ref, st_ref):
    i = pl.program_id(0)
    h = jnp.dot(x1_ref[...], w5_ref[0:64, :], preferred_element_type=jnp.float32)
    h += jnp.dot(x2_ref[...], w5_ref[64:128, :], preferred_element_type=jnp.float32)
    h += jnp.dot(x3_ref[...], w5_ref[128:256, :], preferred_element_type=jnp.float32)
    h += jnp.dot(x4_ref[...], w5_ref[256:512, :], preferred_element_type=jnp.float32)
    h5_ref[...] = h
    part = jnp.concatenate(
        [
            jnp.sum(h, axis=0, keepdims=True),
            jnp.sum(h * h, axis=0, keepdims=True),
            jnp.zeros((6, h.shape[1]), jnp.float32),
        ],
        0,
    )

    @pl.when(i == 0)
    def _():
        st_ref[...] = jnp.zeros_like(st_ref)

    st_ref[...] += part


def _h1_call():
    return _pc(
        _h1_body,
        grid=(NBLK,),
        in_specs=[
            pl.BlockSpec((BLK, 64), lambda i: (i, 0)),
            pl.BlockSpec((BLK, 64), lambda i: (i, 0)),
            pl.BlockSpec((BLK, 128), lambda i: (i, 0)),
            pl.BlockSpec((BLK, 256), lambda i: (i, 0)),
            pl.BlockSpec((512, 512), lambda i: (0, 0)),
        ],
        out_specs=(
            pl.BlockSpec((BLK, 512), lambda i: (i, 0)),
            pl.BlockSpec((8, 512), lambda i: (0, 0)),
        ),
        out_shape=(
            jax.ShapeDtypeStruct((N, 512), jnp.float32),
            jax.ShapeDtypeStruct((8, 512), jnp.float32),
        ),
    )


def _h2_body(h5_ref, st5_ref, gb5_ref, wf_ref, bf_ref, g_ref, stf_ref):
    i = pl.program_id(0)
    st = st5_ref[...]
    mean = st[0:1] * (1.0 / N)
    var = st[1:2] * (1.0 / N) - mean * mean
    a = gb5_ref[0:1] * lax.rsqrt(var + EPS)
    cc = gb5_ref[1:2] - a * mean
    c5 = _lrelu(a * h5_ref[...] + cc)
    g = jnp.dot(c5, wf_ref[...], preferred_element_type=jnp.float32) + bf_ref[0:1]
    g_ref[...] = g
    part = jnp.concatenate(
        [
            jnp.sum(g, axis=0, keepdims=True),
            jnp.sum(g * g, axis=0, keepdims=True),
            jnp.zeros((6, g.shape[1]), jnp.float32),
        ],
        0,
    )

    @pl.when(i == 0)
    def _():
        stf_ref[...] = jnp.zeros_like(stf_ref)

    stf_ref[...] += part


def _h2_call():
    return _pc(
        _h2_body,
        grid=(NBLK,),
        in_specs=[
            pl.BlockSpec((BLK, 512), lambda i: (i, 0)),
            pl.BlockSpec((8, 512), lambda i: (0, 0)),
            pl.BlockSpec((8, 512), lambda i: (0, 0)),
            pl.BlockSpec((512, 1024), lambda i: (0, 0)),
            pl.BlockSpec((8, 1024), lambda i: (0, 0)),
        ],
        out_specs=(
            pl.BlockSpec((BLK, 1024), lambda i: (i, 0)),
            pl.BlockSpec((8, 1024), lambda i: (0, 0)),
        ),
        out_shape=(
            jax.ShapeDtypeStruct((N, 1024), jnp.float32),
            jax.ShapeDtypeStruct((8, 1024), jnp.float32),
        ),
    )


def _h3_body(g_ref, stf_ref, gbf_ref, wsem_ref, bsem_ref, f_ref, sem_ref):
    st = stf_ref[...]
    mean = st[0:1] * (1.0 / N)
    var = st[1:2] * (1.0 / N) - mean * mean
    a = gbf_ref[0:1] * lax.rsqrt(var + EPS)
    cc = gbf_ref[1:2] - a * mean
    f = a * g_ref[...] + cc
    f_ref[...] = f
    sem_ref[...] = (
        jnp.dot(f[:, 768:1024], wsem_ref[...], preferred_element_type=jnp.float32)
        + bsem_ref[0:1]
    )


def _h3_call():
    return _pc(
        _h3_body,
        grid=(NBLK,),
        in_specs=[
            pl.BlockSpec((BLK, 1024), lambda i: (i, 0)),
            pl.BlockSpec((8, 1024), lambda i: (0, 0)),
            pl.BlockSpec((8, 1024), lambda i: (0, 0)),
            pl.BlockSpec((256, 32), lambda i: (0, 0)),
            pl.BlockSpec((8, 32), lambda i: (0, 0)),
        ],
        out_specs=(
            pl.BlockSpec((BLK, 1024), lambda i: (i, 0)),
            pl.BlockSpec((BLK, 32), lambda i: (i, 0)),
        ),
        out_shape=(
            jax.ShapeDtypeStruct((N, 1024), jnp.float32),
            jax.ShapeDtypeStruct((N, 32), jnp.float32),
        ),
    )


def _pad_rows(*vecs):
    d = vecs[0].shape[0]
    rows = [v[None].astype(jnp.float32) for v in vecs]
    rows.append(jnp.zeros((8 - len(vecs), d), jnp.float32))
    return jnp.concatenate(rows, 0)


def kernel(coords, feats, W1, g1, b1, W2, g2, b2, W3, g3, b3, W4, g4, b4,
           W5, g5, b5, Wf5, bf5, gf5, betaf5, Wf6, bf6, gf6, betaf6,
           Wf7, bf7, gf7, betaf7, Wf8, bf8, gf8, betaf8, Wsem, bsem):
    x = jnp.concatenate([coords, feats[:, 3:4]], axis=1)
    outs = []
    for W, g, b in ((W1, g1, b1), (W2, g2, b2), (W3, g3, b3), (W4, g4, b4)):
        c = x.shape[1]
        o = W.shape[0]
        w_t = W.T
        xpad = jnp.pad(x, ((0, 0), (0, CP - c))) if c < CP else x
        idx = _topk_call(c)(x, x, x.T)
        idx_flat = idx[:, :KNN].reshape(-1)
        g_rows = _sc_gather_call()(idx_flat, xpad)
        hmax, hmin, stats = _edge_call(c, o)(g_rows, x, w_t)
        x = _fin_call(o)(stats, _pad_rows(g, b), hmax, hmin)
        outs.append(x)

    x1, x2, x3, x4 = outs
    h5, st5 = _h1_call()(x1, x2, x3, x4, W5.T)
    wf_all = jnp.concatenate([Wf5.T, Wf6.T, Wf7.T, Wf8.T], axis=1)
    bf_all = jnp.concatenate([bf5, bf6, bf7, bf8])
    g_all, stf = _h2_call()(h5, st5, _pad_rows(g5, b5), wf_all, _pad_rows(bf_all))
    gbf = _pad_rows(
        jnp.concatenate([gf5, gf6, gf7, gf8]),
        jnp.concatenate([betaf5, betaf6, betaf7, betaf8]),
    )
    wsem_pad = jnp.concatenate(
        [Wsem.T, jnp.zeros((256, 12), jnp.float32)], axis=1
    )
    bsem_pad = _pad_rows(jnp.concatenate([bsem, jnp.zeros((12,), jnp.float32)]))
    f_all, sem = _h3_call()(g_all, stf, gbf, wsem_pad, bsem_pad)

    ms_features = tuple(
        f_all[:, i * 256 : (i + 1) * 256][None] for i in range(4)
    )
    sem_logits = sem[:, :20][None]
    ms_coords = coords[None]
    ms_masks = jnp.zeros((1, N), dtype=bool)
    return (*ms_features, ms_coords, ms_masks, sem_logits)
